# baseline (speedup 1.0000x reference)
import functools

import jax
import jax.numpy as jnp
from jax import lax
from jax.experimental import pallas as pl
from jax.experimental.pallas import tpu as pltpu
from jax.experimental.pallas import tpu_sc as plsc

B = 16384
H = 50
D = 32
V = 1_000_000
NC, NS = 2, 16
NW = NC * NS
PB = B // NW
IW = 128
ND = PB // IW
NB = 2
PBP = PB + 1
L = 16
C2 = 160
C2P = C2 + 1
NCH = V // C2
NIT = -(-NCH // NS)


@functools.cache
def _build():
    mesh = plsc.VectorSubcoreMesh(
        core_axis_name="c", subcore_axis_name="s",
        num_cores=NC, num_subcores=NS)

    @functools.partial(
        pl.kernel,
        out_type=(
            jax.ShapeDtypeStruct((H, D, B), jnp.float32),
            jax.ShapeDtypeStruct((NC, V, D), jnp.float32),
        ),
        mesh=mesh,
        compiler_params=pltpu.CompilerParams(
            use_tc_tiling_on_sc=False, needs_layout_passes=False),
        scratch_types=[
            pltpu.VMEM((H, PB), jnp.int32),
            pltpu.VMEM((NB, PB, D), jnp.float32),
            pltpu.VMEM((NB, D, PBP), jnp.float32),
            pltpu.VMEM((NB, D, C2P), jnp.float32),
            pltpu.VMEM((NB, C2, D), jnp.float32),
            pltpu.SemaphoreType.DMA,
            pltpu.SemaphoreType.DMA,
            pltpu.SemaphoreType.DMA,
            pltpu.SemaphoreType.DMA,
        ],
    )
    def k(idx_hbm, tt_hbm, out_hbm, tab_hbm,
          idx_v, gbuf, tbuf, slab, rblk, sem_g, sem_out, sem_r, sem_w):
        myc = lax.axis_index("c")
        sid = lax.axis_index("s")
        wid = sid * NC + myc
        base = wid * PB
        pltpu.sync_copy(idx_hbm.at[:, pl.ds(base, PB)], idx_v)
        lanes_lo = lax.iota(jnp.int32, L)
        lanes_hi = lanes_lo + L
        def read_slab(kk, bb):
            ci = kk * NS + sid

            @pl.when(ci < NCH)
            def _():
                pltpu.async_copy(
                    tt_hbm.at[:, pl.ds(ci * C2, C2)],
                    slab.at[bb, :, pl.ds(0, C2)],
                    sem_r)

        for bb in range(NB):
            read_slab(bb, bb)

        @pl.loop(0, NIT)
        def fmt(kk):
            ci = kk * NS + sid

            @pl.when(ci < NCH)
            def _():
                @pl.when(kk > 1)
                def _():
                    pltpu.make_async_copy(
                        rblk.at[0], tab_hbm.at[0, pl.ds(0, C2)], sem_w
                    ).wait()

                for bb in range(NB):
                    @pl.when(lax.rem(kk, 2) == bb)
                    def _():
                        pltpu.make_async_copy(
                            tt_hbm.at[:, pl.ds(0, C2)],
                            slab.at[bb, :, pl.ds(0, C2)],
                            sem_r,
                        ).wait()

                        @plsc.parallel_loop(
                            0, C2, unroll=4,
                            carry=jnp.zeros((L,), jnp.int32))
                        def per_col(c, cvec):
                            v0 = plsc.load_gather(slab.at[bb],
                                                  [lanes_lo, cvec])
                            v1 = plsc.load_gather(slab.at[bb],
                                                  [lanes_hi, cvec])
                            rblk[bb, c, pl.ds(0, L)] = v0
                            rblk[bb, c, pl.ds(L, L)] = v1
                            return cvec + 1

                        pltpu.async_copy(
                            rblk.at[bb],
                            tab_hbm.at[myc, pl.ds(ci * C2, C2)],
                            sem_w)
                        read_slab(kk + 2, bb)

        for _ in range(2):
            pltpu.make_async_copy(
                rblk.at[0], tab_hbm.at[0, pl.ds(0, C2)], sem_w
            ).wait()
        plsc.subcore_barrier()

        @pl.loop(0, H // NB)
        def body(g):
            @pl.when(g > 0)
            def _():
                for b in range(NB):
                    pltpu.make_async_copy(
                        tbuf.at[b, :, pl.ds(0, PB)],
                        out_hbm.at[0, :, pl.ds(0, PB)], sem_out
                    ).wait()

            descs = []
            for b in range(NB):
                h = g * NB + b
                for q in range(ND):
                    d = pltpu.async_copy(
                        tab_hbm.at[myc].at[idx_v.at[h, pl.ds(q * IW, IW)]],
                        gbuf.at[b, pl.ds(q * IW, IW)],
                        sem_g)
                    descs.append(d)
            for b in range(NB):
                for q in range(ND):
                    descs[b * ND + q].wait()

                @plsc.parallel_loop(0, PB, unroll=8,
                                    carry=jnp.zeros((L,), jnp.int32))
                def per_row(r, rvec):
                    v0 = gbuf[b, r, pl.ds(0, L)]
                    v1 = gbuf[b, r, pl.ds(L, L)]
                    plsc.store_scatter(tbuf.at[b], [lanes_lo, rvec], v0)
                    plsc.store_scatter(tbuf.at[b], [lanes_hi, rvec], v1)
                    return rvec + 1

                h = g * NB + b
                pltpu.async_copy(
                    tbuf.at[b, :, pl.ds(0, PB)],
                    out_hbm.at[h, :, pl.ds(base, PB)],
                    sem_out)

        for b in range(NB):
            pltpu.make_async_copy(
                tbuf.at[b, :, pl.ds(0, PB)],
                out_hbm.at[0, :, pl.ds(0, PB)], sem_out
            ).wait()

    return k


def kernel(label_idx, embedding_center):
    idx_t = jnp.transpose(label_idx.astype(jnp.int32))
    tab_t = jnp.transpose(embedding_center)
    out_t, _ = _build()(idx_t, tab_t)
    return jnp.transpose(out_t, (2, 0, 1))


# speedup vs baseline: 4.2511x; 4.2511x over previous
import functools

import jax
import jax.numpy as jnp
from jax import lax
from jax.experimental import pallas as pl
from jax.experimental.pallas import tpu as pltpu
from jax.experimental.pallas import tpu_sc as plsc

B = 16384
H = 50
D = 32
NC, NS = 2, 16
NW = NC * NS
PB = B // NW
IW = 128
ND = PB // IW
NB = 2
PBP = PB + 1
L = 16


@functools.cache
def _build():
    mesh = plsc.VectorSubcoreMesh(
        core_axis_name="c", subcore_axis_name="s",
        num_cores=NC, num_subcores=NS)

    @functools.partial(
        pl.kernel,
        out_type=jax.ShapeDtypeStruct((H, D, B), jnp.float32),
        mesh=mesh,
        compiler_params=pltpu.CompilerParams(
            use_tc_tiling_on_sc=False, needs_layout_passes=False),
        scratch_types=[
            pltpu.VMEM((H, PB), jnp.int32),
            pltpu.VMEM((NB, PB, D), jnp.float32),
            pltpu.VMEM((NB, D, PBP), jnp.float32),
            pltpu.SemaphoreType.DMA,
            pltpu.SemaphoreType.DMA,
        ],
    )
    def k(idx_hbm, table_hbm, out_hbm, idx_v, gbuf, tbuf, sem_g, sem_out):
        wid = lax.axis_index("s") * NC + lax.axis_index("c")
        base = wid * PB
        pltpu.sync_copy(idx_hbm.at[:, pl.ds(base, PB)], idx_v)
        lanes_lo = lax.iota(jnp.int32, L)
        lanes_hi = lanes_lo + L

        def transpose_block(b):
            @plsc.parallel_loop(0, PB, unroll=8,
                                carry=jnp.zeros((L,), jnp.int32))
            def per_row(r, rvec):
                v0 = gbuf[b, r, pl.ds(0, L)]
                v1 = gbuf[b, r, pl.ds(L, L)]
                plsc.store_scatter(tbuf.at[b], [lanes_lo, rvec], v0)
                plsc.store_scatter(tbuf.at[b], [lanes_hi, rvec], v1)
                return rvec + 1

        @pl.loop(0, H // NB)
        def body(g):
            @pl.when(g > 0)
            def _():
                for b in range(NB):
                    pltpu.make_async_copy(
                        tbuf.at[b, :, pl.ds(0, PB)],
                        out_hbm.at[0, :, pl.ds(0, PB)], sem_out
                    ).wait()

            descs = []
            for b in range(NB):
                h = g * NB + b
                for q in range(ND):
                    d = pltpu.async_copy(
                        table_hbm.at[idx_v.at[h, pl.ds(q * IW, IW)]],
                        gbuf.at[b, pl.ds(q * IW, IW)],
                        sem_g)
                    descs.append(d)
            for b in range(NB):
                for q in range(ND):
                    descs[b * ND + q].wait()
                transpose_block(b)
                h = g * NB + b
                pltpu.async_copy(
                    tbuf.at[b, :, pl.ds(0, PB)],
                    out_hbm.at[h, :, pl.ds(base, PB)],
                    sem_out)

        for b in range(NB):
            pltpu.make_async_copy(
                tbuf.at[b, :, pl.ds(0, PB)],
                out_hbm.at[0, :, pl.ds(0, PB)], sem_out
            ).wait()

    return k


def kernel(label_idx, embedding_center):
    idx_t = jnp.transpose(label_idx.astype(jnp.int32))
    out_t = _build()(idx_t, embedding_center)
    return jnp.transpose(out_t, (2, 0, 1))
